# Initial kernel scaffold; baseline (speedup 1.0000x reference)
import jax
import jax.numpy as jnp
from jax.experimental import pallas as pl


def kernel(x, wb_coarse, wb_mid, wb_fine, Wq, bq, Wk, bk, Wv, bv, Wo, bo):
    raise NotImplementedError("write your pallas kernel here")


# speedup vs baseline: 5.3405x; 5.3405x over previous
import functools
import math

import jax
import jax.numpy as jnp
from jax import lax
from jax.experimental import pallas as pl
from jax.experimental.pallas import tpu as pltpu
from jax.experimental.pallas import tpu_sc as plsc

N = 4096
E = 1024
H = 16
DH = 64
NB = 256
RB = 256
LEVELS = ((256, 256), (64, 64), (16, 64))


def _rank_body(keys_ref, out_ref):
    keys = keys_ref[0]
    cols = lax.broadcasted_iota(jnp.int32, (N, NB), 1)
    onehot_b = keys == cols
    onehot = onehot_b.astype(jnp.bfloat16)

    rows_i = lax.broadcasted_iota(jnp.int32, (RB, RB), 0)
    cols_i = lax.broadcasted_iota(jnp.int32, (RB, RB), 1)
    lstrict = (cols_i < rows_i).astype(jnp.bfloat16)
    urows = lax.broadcasted_iota(jnp.int32, (NB, NB), 0)
    ucols = lax.broadcasted_iota(jnp.int32, (NB, NB), 1)
    ustrict = (urows < ucols).astype(jnp.bfloat16)

    nblk = N // RB
    cums, hists = [], []
    for b in range(nblk):
        blk = onehot[b * RB:(b + 1) * RB]
        cums.append(lax.dot_general(lstrict, blk, (((1,), (0,)), ((), ())),
                                    preferred_element_type=jnp.float32))
        hists.append(jnp.sum(blk.astype(jnp.float32), axis=0, keepdims=True))

    total = jnp.zeros((1, NB), jnp.float32)
    offs = []
    for b in range(nblk):
        offs.append(total)
        total = total + hists[b]

    t_hi = jnp.floor(total * (1.0 / 16.0))
    t_lo = total - t_hi * 16.0
    off_hi = lax.dot_general(t_hi.astype(jnp.bfloat16), ustrict,
                             (((1,), (0,)), ((), ())),
                             preferred_element_type=jnp.float32)
    off_lo = lax.dot_general(t_lo.astype(jnp.bfloat16), ustrict,
                             (((1,), (0,)), ((), ())),
                             preferred_element_type=jnp.float32)
    offset = off_hi * 16.0 + off_lo

    for b in range(nblk):
        blk_b = onehot_b[b * RB:(b + 1) * RB]
        vals = cums[b] + offs[b] + offset
        r = jnp.sum(jnp.where(blk_b, vals, 0.0), axis=1, keepdims=True)
        out_ref[0, b * RB:(b + 1) * RB] = r.astype(jnp.int32)


def _ranks(wb3):
    return pl.pallas_call(
        _rank_body,
        grid=(3,),
        in_specs=[pl.BlockSpec((1, N, 1), lambda l: (l, 0, 0))],
        out_specs=pl.BlockSpec((1, N, 1), lambda l: (l, 0, 0)),
        out_shape=jax.ShapeDtypeStruct((3, N, 1), jnp.int32),
    )(wb3)


def _mm_body(x_ref, w_ref, b_ref, o_ref, *, scale):
    xb = x_ref[...].astype(jnp.bfloat16)
    wb = w_ref[...].astype(jnp.bfloat16)
    acc = jnp.dot(xb, wb, preferred_element_type=jnp.float32)
    if scale != 1.0:
        acc = acc * scale
    o_ref[...] = acc + b_ref[0]


def _matmul(x, w, b3, scale=1.0, br=512, bc=1024):
    m, k = x.shape
    _, n = w.shape
    return pl.pallas_call(
        functools.partial(_mm_body, scale=scale),
        grid=(n // bc, m // br),
        in_specs=[
            pl.BlockSpec((br, k), lambda c, r: (r, 0)),
            pl.BlockSpec((k, bc), lambda c, r: (0, c)),
            pl.BlockSpec((1, 1, bc), lambda c, r: (0, 0, c)),
        ],
        out_specs=pl.BlockSpec((br, bc), lambda c, r: (r, c)),
        out_shape=jax.ShapeDtypeStruct((m, n), jnp.float32),
    )(x, w, b3)


def _sum_mm_body(a0_ref, a1_ref, a2_ref, w_ref, b_ref, o_ref, *, scale):
    s = a0_ref[...] + a1_ref[...] + a2_ref[...]
    sb = s.astype(jnp.bfloat16)
    wb = w_ref[...].astype(jnp.bfloat16)
    acc = jnp.dot(sb, wb, preferred_element_type=jnp.float32)
    o_ref[...] = acc * scale + b_ref[0]


def _final_mm(outs_u, w, b3, br=512):
    nr = N // br
    return pl.pallas_call(
        functools.partial(_sum_mm_body, scale=1.0 / 3.0),
        grid=(nr,),
        in_specs=[
            pl.BlockSpec((br, E), lambda r: (r, 0)),
            pl.BlockSpec((br, E), lambda r: (nr + r, 0)),
            pl.BlockSpec((br, E), lambda r: (2 * nr + r, 0)),
            pl.BlockSpec((E, E), lambda r: (0, 0)),
            pl.BlockSpec((1, 1, E), lambda r: (0, 0, 0)),
        ],
        out_specs=pl.BlockSpec((br, E), lambda r: (r, 0)),
        out_shape=jax.ShapeDtypeStruct((N, E), jnp.float32),
    )(outs_u, outs_u, outs_u, w, b3)


def _attn_body(q_ref, k1_ref, k2_ref, v1_ref, v2_ref, o_ref, *, C, QB):
    WIN = 2 * QB
    qb = pl.program_id(1)
    q2 = q_ref[...].astype(jnp.bfloat16)
    kcat2 = jnp.concatenate([k1_ref[...], k2_ref[...]], axis=0).astype(jnp.bfloat16)
    vcat2 = jnp.concatenate([v1_ref[...], v2_ref[...]], axis=0).astype(jnp.bfloat16)
    p_row = qb * QB + lax.broadcasted_iota(jnp.int32, (QB, WIN), 0)
    kpos = (qb - 1) * QB + lax.broadcasted_iota(jnp.int32, (QB, WIN), 1)
    c = p_row // C
    mask = (kpos >= c * C - C) & (kpos < c * C + C) & (kpos >= 0)
    outs = []
    for i in range(2):
        q = q2[:, i * DH:(i + 1) * DH]
        kcat = kcat2[:, i * DH:(i + 1) * DH]
        vcat = vcat2[:, i * DH:(i + 1) * DH]
        s = lax.dot_general(q, kcat, (((1,), (1,)), ((), ())),
                            preferred_element_type=jnp.float32)
        s = s * (1.0 / math.sqrt(DH))
        s = jnp.where(mask, s, -1e30)
        m = jnp.max(s, axis=1, keepdims=True)
        e = jnp.exp(s - m)
        denom = jnp.sum(e, axis=1, keepdims=True)
        p = (e / denom).astype(jnp.bfloat16)
        outs.append(lax.dot_general(p, vcat, (((1,), (0,)), ((), ())),
                                    preferred_element_type=jnp.float32))
    o_ref[...] = jnp.concatenate(outs, axis=1)


def _attn(qkv_s, C, QB):
    nqb = N // QB
    hp = H // 2
    prev = lambda qb: jnp.maximum(qb - 1, 0)
    return pl.pallas_call(
        functools.partial(_attn_body, C=C, QB=QB),
        grid=(hp, nqb),
        in_specs=[
            pl.BlockSpec((QB, 2 * DH), lambda h, qb: (qb, h)),
            pl.BlockSpec((QB, 2 * DH), lambda h, qb: (prev(qb), hp + h)),
            pl.BlockSpec((QB, 2 * DH), lambda h, qb: (qb, hp + h)),
            pl.BlockSpec((QB, 2 * DH), lambda h, qb: (prev(qb), 2 * hp + h)),
            pl.BlockSpec((QB, 2 * DH), lambda h, qb: (qb, 2 * hp + h)),
        ],
        out_specs=pl.BlockSpec((QB, 2 * DH), lambda h, qb: (qb, h)),
        out_shape=jax.ShapeDtypeStruct((N, E), jnp.float32),
    )(qkv_s, qkv_s, qkv_s, qkv_s, qkv_s)


def _sc_sort3(qkv, ranks):
    info = plsc.get_sparse_core_info()
    nw = info.num_cores * info.num_subcores
    rpt = N // nw
    ch = 32
    nch = rpt // ch
    mesh = plsc.VectorSubcoreMesh(core_axis_name="c", subcore_axis_name="s")

    @functools.partial(
        pl.kernel, mesh=mesh,
        out_type=tuple(jax.ShapeDtypeStruct((N, 3 * E), jnp.float32)
                       for _ in range(3)),
        scratch_types=[
            pltpu.VMEM((ch, 3 * E), jnp.float32),
            pltpu.VMEM((3 * nch, ch), jnp.int32),
            pltpu.SemaphoreType.DMA,
        ],
    )
    def k(qkv_hbm, ranks_hbm, o0, o1, o2, buf, idx_v, sem):
        outs = (o0, o1, o2)
        wid = lax.axis_index("s") * info.num_cores + lax.axis_index("c")
        base = wid * rpt
        for l in range(3):
            for c in range(nch):
                pltpu.sync_copy(ranks_hbm.at[l, pl.ds(base + c * ch, ch)],
                                idx_v.at[l * nch + c])
        for c in range(nch):
            pltpu.sync_copy(qkv_hbm.at[pl.ds(base + c * ch, ch)], buf)
            handles = [
                pltpu.async_copy(buf, outs[l].at[idx_v.at[l * nch + c]], sem)
                for l in range(3)
            ]
            for h_ in handles:
                h_.wait()

    return k(qkv, ranks)


def _sc_unsort3(a0, a1, a2, ranks):
    info = plsc.get_sparse_core_info()
    nw = info.num_cores * info.num_subcores
    rpt = N // nw
    ch = 32
    nch = rpt // ch
    mesh = plsc.VectorSubcoreMesh(core_axis_name="c", subcore_axis_name="s")

    @functools.partial(
        pl.kernel, mesh=mesh,
        out_type=jax.ShapeDtypeStruct((3 * N, E), jnp.float32),
        scratch_types=[
            pltpu.VMEM((ch, E), jnp.float32),
            pltpu.VMEM((ch, E), jnp.float32),
            pltpu.VMEM((ch, E), jnp.float32),
            pltpu.VMEM((3 * nch, ch), jnp.int32),
            pltpu.SemaphoreType.DMA,
        ],
    )
    def k(in0, in1, in2, ranks_hbm, out_hbm, b0, b1, b2, idx_v, sem):
        ins = (in0, in1, in2)
        bufs = (b0, b1, b2)
        wid = lax.axis_index("s") * info.num_cores + lax.axis_index("c")
        base = wid * rpt
        for l in range(3):
            for c in range(nch):
                pltpu.sync_copy(ranks_hbm.at[l, pl.ds(base + c * ch, ch)],
                                idx_v.at[l * nch + c])
        for c in range(nch):
            handles = [
                pltpu.async_copy(ins[l].at[idx_v.at[l * nch + c]], bufs[l], sem)
                for l in range(3)
            ]
            for h_ in handles:
                h_.wait()
            for l in range(3):
                pltpu.sync_copy(bufs[l],
                                out_hbm.at[pl.ds(l * N + base + c * ch, ch)])

    return k(a0, a1, a2, ranks)


def kernel(x, wb_coarse, wb_mid, wb_fine, Wq, bq, Wk, bk, Wv, bv, Wo, bo):
    x2 = x.reshape(N, E)
    wb3 = jnp.stack([wb_coarse, wb_mid, wb_fine]).astype(jnp.int32)
    ranks = _ranks(wb3.reshape(3, N, 1)).reshape(3, N)
    Wqkv = jnp.concatenate([Wq, Wk, Wv], axis=1)
    bqkv = jnp.concatenate([bq, bk, bv]).reshape(1, 1, 3 * E)
    qkv = _matmul(x2, Wqkv, bqkv)
    sorted_lvls = _sc_sort3(qkv, ranks)
    outs_sorted = [_attn(s_l, C, QB)
                   for s_l, (C, QB) in zip(sorted_lvls, LEVELS)]
    outs_u = _sc_unsort3(*outs_sorted, ranks)
    y = _final_mm(outs_u, Wo, bo.reshape(1, 1, E))
    return y.reshape(1, N, E)


# speedup vs baseline: 10.0219x; 1.8766x over previous
import functools
import math

import jax
import jax.numpy as jnp
from jax import lax
from jax.experimental import pallas as pl
from jax.experimental.pallas import tpu as pltpu
from jax.experimental.pallas import tpu_sc as plsc

N = 4096
E = 1024
H = 16
DH = 64
NB = 256
RB = 256
LEVELS = ((256, 256), (64, 256), (16, 256))


def _rank_body(keys_ref, out_ref):
    keys = keys_ref[0]
    cols = lax.broadcasted_iota(jnp.int32, (N, NB), 1)
    onehot_b = keys == cols
    onehot = onehot_b.astype(jnp.bfloat16)

    rows_i = lax.broadcasted_iota(jnp.int32, (RB, RB), 0)
    cols_i = lax.broadcasted_iota(jnp.int32, (RB, RB), 1)
    lstrict = (cols_i < rows_i).astype(jnp.bfloat16)
    urows = lax.broadcasted_iota(jnp.int32, (NB, NB), 0)
    ucols = lax.broadcasted_iota(jnp.int32, (NB, NB), 1)
    ustrict = (urows < ucols).astype(jnp.bfloat16)

    nblk = N // RB
    cums, hists = [], []
    for b in range(nblk):
        blk = onehot[b * RB:(b + 1) * RB]
        cums.append(lax.dot_general(lstrict, blk, (((1,), (0,)), ((), ())),
                                    preferred_element_type=jnp.float32))
        hists.append(jnp.sum(blk.astype(jnp.float32), axis=0, keepdims=True))

    total = jnp.zeros((1, NB), jnp.float32)
    offs = []
    for b in range(nblk):
        offs.append(total)
        total = total + hists[b]

    t_hi = jnp.floor(total * (1.0 / 16.0))
    t_lo = total - t_hi * 16.0
    off_hi = lax.dot_general(t_hi.astype(jnp.bfloat16), ustrict,
                             (((1,), (0,)), ((), ())),
                             preferred_element_type=jnp.float32)
    off_lo = lax.dot_general(t_lo.astype(jnp.bfloat16), ustrict,
                             (((1,), (0,)), ((), ())),
                             preferred_element_type=jnp.float32)
    offset = off_hi * 16.0 + off_lo

    for b in range(nblk):
        blk_b = onehot_b[b * RB:(b + 1) * RB]
        vals = cums[b] + offs[b] + offset
        r = jnp.sum(jnp.where(blk_b, vals, 0.0), axis=1, keepdims=True)
        out_ref[0, b * RB:(b + 1) * RB] = r.astype(jnp.int32)


def _ranks(wb3):
    return pl.pallas_call(
        _rank_body,
        grid=(3,),
        in_specs=[pl.BlockSpec((1, N, 1), lambda l: (l, 0, 0))],
        out_specs=pl.BlockSpec((1, N, 1), lambda l: (l, 0, 0)),
        out_shape=jax.ShapeDtypeStruct((3, N, 1), jnp.int32),
    )(wb3)


def _mm_body(x_ref, w_ref, b_ref, o_ref, *, scale):
    xb = x_ref[...].astype(jnp.bfloat16)
    wb = w_ref[...].astype(jnp.bfloat16)
    acc = jnp.dot(xb, wb, preferred_element_type=jnp.float32)
    if scale != 1.0:
        acc = acc * scale
    o_ref[...] = acc + b_ref[0]


def _matmul(x, w, b3, scale=1.0, br=512, bc=1024):
    m, k = x.shape
    _, n = w.shape
    return pl.pallas_call(
        functools.partial(_mm_body, scale=scale),
        grid=(n // bc, m // br),
        in_specs=[
            pl.BlockSpec((br, k), lambda c, r: (r, 0)),
            pl.BlockSpec((k, bc), lambda c, r: (0, c)),
            pl.BlockSpec((1, 1, bc), lambda c, r: (0, 0, c)),
        ],
        out_specs=pl.BlockSpec((br, bc), lambda c, r: (r, c)),
        out_shape=jax.ShapeDtypeStruct((m, n), jnp.float32),
    )(x, w, b3)


def _sum_mm_body(a0_ref, a1_ref, a2_ref, w_ref, b_ref, o_ref, *, scale):
    s = a0_ref[...] + a1_ref[...] + a2_ref[...]
    sb = s.astype(jnp.bfloat16)
    wb = w_ref[...].astype(jnp.bfloat16)
    acc = jnp.dot(sb, wb, preferred_element_type=jnp.float32)
    o_ref[...] = acc * scale + b_ref[0]


def _final_mm(outs_u, w, b3, br=512):
    nr = N // br
    return pl.pallas_call(
        functools.partial(_sum_mm_body, scale=1.0 / 3.0),
        grid=(nr,),
        in_specs=[
            pl.BlockSpec((br, E), lambda r: (r, 0)),
            pl.BlockSpec((br, E), lambda r: (nr + r, 0)),
            pl.BlockSpec((br, E), lambda r: (2 * nr + r, 0)),
            pl.BlockSpec((E, E), lambda r: (0, 0)),
            pl.BlockSpec((1, 1, E), lambda r: (0, 0, 0)),
        ],
        out_specs=pl.BlockSpec((br, E), lambda r: (r, 0)),
        out_shape=jax.ShapeDtypeStruct((N, E), jnp.float32),
    )(outs_u, outs_u, outs_u, w, b3)


def _attn_body(q_ref, k1_ref, k2_ref, v1_ref, v2_ref, o_ref, *, C, QB):
    WIN = C + QB
    qb = pl.program_id(1)
    q2 = q_ref[...].astype(jnp.bfloat16)
    kcat2 = jnp.concatenate([k1_ref[...], k2_ref[...]], axis=0).astype(jnp.bfloat16)
    vcat2 = jnp.concatenate([v1_ref[...], v2_ref[...]], axis=0).astype(jnp.bfloat16)
    p_row = qb * QB + lax.broadcasted_iota(jnp.int32, (QB, WIN), 0)
    kpos = qb * QB - C + lax.broadcasted_iota(jnp.int32, (QB, WIN), 1)
    c = p_row // C
    mask = (kpos >= c * C - C) & (kpos < c * C + C) & (kpos >= 0)
    outs = []
    for i in range(2):
        q = q2[:, i * DH:(i + 1) * DH]
        kcat = kcat2[:, i * DH:(i + 1) * DH]
        vcat = vcat2[:, i * DH:(i + 1) * DH]
        s = lax.dot_general(q, kcat, (((1,), (1,)), ((), ())),
                            preferred_element_type=jnp.float32)
        s = s * (1.0 / math.sqrt(DH))
        s = jnp.where(mask, s, -1e30)
        m = jnp.max(s, axis=1, keepdims=True)
        e = jnp.exp(s - m)
        denom = jnp.sum(e, axis=1, keepdims=True)
        p = (e / denom).astype(jnp.bfloat16)
        outs.append(lax.dot_general(p, vcat, (((1,), (0,)), ((), ())),
                                    preferred_element_type=jnp.float32))
    o_ref[...] = jnp.concatenate(outs, axis=1)


def _attn(qkv_s, C, QB):
    nqb = N // QB
    hp = H // 2
    r = QB // C
    halo = lambda qb: jnp.maximum(qb * r - 1, 0)
    return pl.pallas_call(
        functools.partial(_attn_body, C=C, QB=QB),
        grid=(hp, nqb),
        in_specs=[
            pl.BlockSpec((QB, 2 * DH), lambda h, qb: (qb, h)),
            pl.BlockSpec((C, 2 * DH), lambda h, qb: (halo(qb), hp + h)),
            pl.BlockSpec((QB, 2 * DH), lambda h, qb: (qb, hp + h)),
            pl.BlockSpec((C, 2 * DH), lambda h, qb: (halo(qb), 2 * hp + h)),
            pl.BlockSpec((QB, 2 * DH), lambda h, qb: (qb, 2 * hp + h)),
        ],
        out_specs=pl.BlockSpec((QB, 2 * DH), lambda h, qb: (qb, h)),
        out_shape=jax.ShapeDtypeStruct((N, E), jnp.float32),
    )(qkv_s, qkv_s, qkv_s, qkv_s, qkv_s)


def _sc_sort3(qkv, ranks):
    info = plsc.get_sparse_core_info()
    nw = info.num_cores * info.num_subcores
    rpt = N // nw
    ch = 32
    nch = rpt // ch
    mesh = plsc.VectorSubcoreMesh(core_axis_name="c", subcore_axis_name="s")

    @functools.partial(
        pl.kernel, mesh=mesh,
        out_type=tuple(jax.ShapeDtypeStruct((N, 3 * E), jnp.float32)
                       for _ in range(3)),
        scratch_types=[
            pltpu.VMEM((ch, 3 * E), jnp.float32),
            pltpu.VMEM((3 * nch, ch), jnp.int32),
            pltpu.SemaphoreType.DMA,
        ],
    )
    def k(qkv_hbm, ranks_hbm, o0, o1, o2, buf, idx_v, sem):
        outs = (o0, o1, o2)
        wid = lax.axis_index("s") * info.num_cores + lax.axis_index("c")
        base = wid * rpt
        for l in range(3):
            for c in range(nch):
                pltpu.sync_copy(ranks_hbm.at[l, pl.ds(base + c * ch, ch)],
                                idx_v.at[l * nch + c])
        for c in range(nch):
            pltpu.sync_copy(qkv_hbm.at[pl.ds(base + c * ch, ch)], buf)
            handles = [
                pltpu.async_copy(buf, outs[l].at[idx_v.at[l * nch + c]], sem)
                for l in range(3)
            ]
            for h_ in handles:
                h_.wait()

    return k(qkv, ranks)


def _sc_unsort3(a0, a1, a2, ranks):
    info = plsc.get_sparse_core_info()
    nw = info.num_cores * info.num_subcores
    rpt = N // nw
    ch = 32
    nch = rpt // ch
    mesh = plsc.VectorSubcoreMesh(core_axis_name="c", subcore_axis_name="s")

    @functools.partial(
        pl.kernel, mesh=mesh,
        out_type=jax.ShapeDtypeStruct((3 * N, E), jnp.float32),
        scratch_types=[
            pltpu.VMEM((ch, E), jnp.float32),
            pltpu.VMEM((ch, E), jnp.float32),
            pltpu.VMEM((ch, E), jnp.float32),
            pltpu.VMEM((3 * nch, ch), jnp.int32),
            pltpu.SemaphoreType.DMA,
        ],
    )
    def k(in0, in1, in2, ranks_hbm, out_hbm, b0, b1, b2, idx_v, sem):
        ins = (in0, in1, in2)
        bufs = (b0, b1, b2)
        wid = lax.axis_index("s") * info.num_cores + lax.axis_index("c")
        base = wid * rpt
        for l in range(3):
            for c in range(nch):
                pltpu.sync_copy(ranks_hbm.at[l, pl.ds(base + c * ch, ch)],
                                idx_v.at[l * nch + c])
        for c in range(nch):
            handles = [
                pltpu.async_copy(ins[l].at[idx_v.at[l * nch + c]], bufs[l], sem)
                for l in range(3)
            ]
            for h_ in handles:
                h_.wait()
            for l in range(3):
                pltpu.sync_copy(bufs[l],
                                out_hbm.at[pl.ds(l * N + base + c * ch, ch)])

    return k(a0, a1, a2, ranks)


def kernel(x, wb_coarse, wb_mid, wb_fine, Wq, bq, Wk, bk, Wv, bv, Wo, bo):
    x2 = x.reshape(N, E)
    wb3 = jnp.stack([wb_coarse, wb_mid, wb_fine]).astype(jnp.int32)
    ranks = _ranks(wb3.reshape(3, N, 1)).reshape(3, N)
    Wqkv = jnp.concatenate([Wq, Wk, Wv], axis=1)
    bqkv = jnp.concatenate([bq, bk, bv]).reshape(1, 1, 3 * E)
    qkv = _matmul(x2, Wqkv, bqkv)
    sorted_lvls = _sc_sort3(qkv, ranks)
    outs_sorted = [_attn(s_l, C, QB)
                   for s_l, (C, QB) in zip(sorted_lvls, LEVELS)]
    outs_u = _sc_unsort3(*outs_sorted, ranks)
    y = _final_mm(outs_u, Wo, bo.reshape(1, 1, E))
    return y.reshape(1, N, E)
